# baseline (speedup 1.0000x reference)
import functools

import jax
import jax.numpy as jnp
from jax import lax
from jax.experimental import pallas as pl
from jax.experimental.pallas import tpu as pltpu
import jax.experimental.pallas.tpu_sc as plsc

ROWS = 8192
DIM = 1024
NUM_CORES = 2
NUM_SUBCORES = 16
NUM_WORKERS = NUM_CORES * NUM_SUBCORES

SC_ROWS = 2048
TC_ROWS = ROWS - SC_ROWS
ROWS_PER_WORKER = SC_ROWS // NUM_WORKERS
CHUNK = 32
CHUNKS = [CHUNK] * (ROWS_PER_WORKER // CHUNK)
OFFS = [sum(CHUNKS[:i]) for i in range(len(CHUNKS))]
NCHUNKS = len(CHUNKS)
NBUF = 2
BLK = 512


@functools.partial(
    pl.kernel,
    out_type=jax.ShapeDtypeStruct((SC_ROWS, DIM), jnp.float32),
    mesh=plsc.VectorSubcoreMesh(core_axis_name="c", subcore_axis_name="s"),
    scratch_types=(
        [pltpu.VMEM((NBUF, CHUNK, DIM), jnp.float32)]
        + [pltpu.SemaphoreType.DMA] * (2 * NBUF)
    ),
)
def _pe_sc(pe_hbm, out_hbm, buf, *sems):
    wid = lax.axis_index("s") * NUM_CORES + lax.axis_index("c")
    base = wid * ROWS_PER_WORKER
    gsems = sems[:NBUF]
    ssems = sems[NBUF:]

    def issue_gather(i):
        return pltpu.async_copy(
            pe_hbm.at[pl.ds(base + OFFS[i], CHUNKS[i])],
            buf.at[i % NBUF, pl.ds(0, CHUNKS[i])],
            gsems[i % NBUF])

    def issue_scatter(i):
        return pltpu.async_copy(
            buf.at[i % NBUF, pl.ds(0, CHUNKS[i])],
            out_hbm.at[pl.ds(base + OFFS[i], CHUNKS[i])],
            ssems[i % NBUF])

    gath = [None] * NCHUNKS
    scat = [None] * NCHUNKS
    gath[0] = issue_gather(0)
    for i in range(NCHUNKS):
        if i + 1 < NCHUNKS:
            if i + 1 >= NBUF:
                scat[i + 1 - NBUF].wait()
            gath[i + 1] = issue_gather(i + 1)
        gath[i].wait()
        scat[i] = issue_scatter(i)
    for i in range(max(0, NCHUNKS - NBUF), NCHUNKS):
        scat[i].wait()


def _copy_body(src_ref, o_ref):
    o_ref[...] = src_ref[...]


def _merge_body(sc_ref, big_ref, o_ref):
    del big_ref
    o_ref[...] = sc_ref[...]


def kernel(x, pe):
    del x
    sc_out = _pe_sc(pe)
    big = pl.pallas_call(
        _copy_body,
        grid=(TC_ROWS // BLK,),
        in_specs=[pl.BlockSpec((BLK, DIM), lambda i: (i + SC_ROWS // BLK, 0))],
        out_specs=pl.BlockSpec((BLK, DIM), lambda i: (i + SC_ROWS // BLK, 0)),
        out_shape=jax.ShapeDtypeStruct((ROWS, DIM), jnp.float32),
    )(pe)
    return pl.pallas_call(
        _merge_body,
        grid=(SC_ROWS // BLK,),
        in_specs=[
            pl.BlockSpec((BLK, DIM), lambda i: (i, 0)),
            pl.BlockSpec(memory_space=pl.ANY),
        ],
        out_specs=pl.BlockSpec((BLK, DIM), lambda i: (i, 0)),
        out_shape=jax.ShapeDtypeStruct((ROWS, DIM), jnp.float32),
        input_output_aliases={1: 0},
    )(sc_out, big)


# speedup vs baseline: 1.0002x; 1.0002x over previous
import functools

import jax
import jax.numpy as jnp
from jax import lax
from jax.experimental import pallas as pl
from jax.experimental.pallas import tpu as pltpu
import jax.experimental.pallas.tpu_sc as plsc

ROWS = 8192
DIM = 1024
NUM_CORES = 2
NUM_SUBCORES = 16
NUM_WORKERS = NUM_CORES * NUM_SUBCORES

SC_ROWS = 2048
TC_ROWS = ROWS - SC_ROWS
ROWS_PER_WORKER = SC_ROWS // NUM_WORKERS
CHUNK = 32
CHUNKS = [CHUNK] * (ROWS_PER_WORKER // CHUNK)
OFFS = [sum(CHUNKS[:i]) for i in range(len(CHUNKS))]
NCHUNKS = len(CHUNKS)
NBUF = 2
BLK = 512


@functools.partial(
    pl.kernel,
    out_type=jax.ShapeDtypeStruct((SC_ROWS, DIM), jnp.float32),
    mesh=plsc.VectorSubcoreMesh(core_axis_name="c", subcore_axis_name="s"),
    scratch_types=(
        [pltpu.VMEM((NBUF, CHUNK, DIM), jnp.float32)]
        + [pltpu.SemaphoreType.DMA] * (2 * NBUF)
    ),
)
def _pe_sc(pe_hbm, out_hbm, buf, *sems):
    wid = lax.axis_index("s") * NUM_CORES + lax.axis_index("c")
    base = wid * ROWS_PER_WORKER
    gsems = sems[:NBUF]
    ssems = sems[NBUF:]

    def issue_gather(i):
        return pltpu.async_copy(
            pe_hbm.at[pl.ds(base + OFFS[i], CHUNKS[i])],
            buf.at[i % NBUF, pl.ds(0, CHUNKS[i])],
            gsems[i % NBUF])

    def issue_scatter(i):
        return pltpu.async_copy(
            buf.at[i % NBUF, pl.ds(0, CHUNKS[i])],
            out_hbm.at[pl.ds(base + OFFS[i], CHUNKS[i])],
            ssems[i % NBUF])

    gath = [None] * NCHUNKS
    scat = [None] * NCHUNKS
    gath[0] = issue_gather(0)
    for i in range(NCHUNKS):
        if i + 1 < NCHUNKS:
            if i + 1 >= NBUF:
                scat[i + 1 - NBUF].wait()
            gath[i + 1] = issue_gather(i + 1)
        gath[i].wait()
        scat[i] = issue_scatter(i)
    for i in range(max(0, NCHUNKS - NBUF), NCHUNKS):
        scat[i].wait()


def _copy_body(src_ref, o_ref):
    o_ref[...] = src_ref[...]


def _merge_body(sc_ref, big_ref, o_ref):
    del big_ref
    o_ref[...] = sc_ref[...]


def kernel(x, pe):
    del x
    big = pl.pallas_call(
        _copy_body,
        grid=(TC_ROWS // BLK,),
        in_specs=[pl.BlockSpec((BLK, DIM), lambda i: (i + SC_ROWS // BLK, 0))],
        out_specs=pl.BlockSpec((BLK, DIM), lambda i: (i + SC_ROWS // BLK, 0)),
        out_shape=jax.ShapeDtypeStruct((ROWS, DIM), jnp.float32),
    )(pe)
    sc_out = _pe_sc(pe)
    return pl.pallas_call(
        _merge_body,
        grid=(SC_ROWS // BLK,),
        in_specs=[
            pl.BlockSpec((BLK, DIM), lambda i: (i, 0)),
            pl.BlockSpec(memory_space=pl.ANY),
        ],
        out_specs=pl.BlockSpec((BLK, DIM), lambda i: (i, 0)),
        out_shape=jax.ShapeDtypeStruct((ROWS, DIM), jnp.float32),
        input_output_aliases={1: 0},
    )(sc_out, big)


# speedup vs baseline: 1.1192x; 1.1190x over previous
import functools

import jax
import jax.numpy as jnp
from jax import lax
from jax.experimental import pallas as pl
from jax.experimental.pallas import tpu as pltpu
import jax.experimental.pallas.tpu_sc as plsc

ROWS = 8192
DIM = 1024
NUM_CORES = 2
NUM_SUBCORES = 16
NUM_WORKERS = NUM_CORES * NUM_SUBCORES
ROWS_PER_WORKER = ROWS // NUM_WORKERS
CHUNKS = [64, 56, 64, 56, 16]
assert sum(CHUNKS) == ROWS_PER_WORKER
OFFS = [sum(CHUNKS[:i]) for i in range(len(CHUNKS))]
NCHUNKS = len(CHUNKS)
NBUF = 2
BUFROWS = (64, 56)


@functools.partial(
    pl.kernel,
    out_type=jax.ShapeDtypeStruct((ROWS, DIM), jnp.float32),
    mesh=plsc.VectorSubcoreMesh(core_axis_name="c", subcore_axis_name="s"),
    scratch_types=(
        [pltpu.VMEM((BUFROWS[0], DIM), jnp.float32),
         pltpu.VMEM((BUFROWS[1], DIM), jnp.float32)]
        + [pltpu.SemaphoreType.DMA] * (2 * NBUF)
    ),
)
def _pe_lookup(pe_hbm, out_hbm, buf0, buf1, *sems):
    wid = lax.axis_index("s") * NUM_CORES + lax.axis_index("c")
    base = wid * ROWS_PER_WORKER
    bufs = (buf0, buf1)
    gsems = sems[:NBUF]
    ssems = sems[NBUF:]

    def issue_gather(i):
        return pltpu.async_copy(
            pe_hbm.at[pl.ds(base + OFFS[i], CHUNKS[i])],
            bufs[i % NBUF].at[pl.ds(0, CHUNKS[i])],
            gsems[i % NBUF])

    def issue_scatter(i):
        return pltpu.async_copy(
            bufs[i % NBUF].at[pl.ds(0, CHUNKS[i])],
            out_hbm.at[pl.ds(base + OFFS[i], CHUNKS[i])],
            ssems[i % NBUF])

    gath = [None] * NCHUNKS
    scat = [None] * NCHUNKS
    gath[0] = issue_gather(0)
    for i in range(NCHUNKS):
        if i + 1 < NCHUNKS:
            if i + 1 >= NBUF:
                scat[i + 1 - NBUF].wait()
            gath[i + 1] = issue_gather(i + 1)
        gath[i].wait()
        scat[i] = issue_scatter(i)
    for i in range(max(0, NCHUNKS - NBUF), NCHUNKS):
        scat[i].wait()


def kernel(x, pe):
    del x
    return _pe_lookup(pe)


# speedup vs baseline: 1.1277x; 1.0075x over previous
import functools

import jax
import jax.numpy as jnp
from jax import lax
from jax.experimental import pallas as pl
from jax.experimental.pallas import tpu as pltpu
import jax.experimental.pallas.tpu_sc as plsc

ROWS = 8192
DIM = 1024
NUM_CORES = 2
NUM_SUBCORES = 16
NUM_WORKERS = NUM_CORES * NUM_SUBCORES
ROWS_PER_WORKER = ROWS // NUM_WORKERS
CHUNKS = [16, 64, 56, 64, 56]
assert sum(CHUNKS) == ROWS_PER_WORKER
OFFS = [sum(CHUNKS[:i]) for i in range(len(CHUNKS))]
NCHUNKS = len(CHUNKS)
NBUF = 2
BUFROWS = (56, 64)


@functools.partial(
    pl.kernel,
    out_type=jax.ShapeDtypeStruct((ROWS, DIM), jnp.float32),
    mesh=plsc.VectorSubcoreMesh(core_axis_name="c", subcore_axis_name="s"),
    scratch_types=(
        [pltpu.VMEM((BUFROWS[0], DIM), jnp.float32),
         pltpu.VMEM((BUFROWS[1], DIM), jnp.float32)]
        + [pltpu.SemaphoreType.DMA] * (2 * NBUF)
    ),
)
def _pe_lookup(pe_hbm, out_hbm, buf0, buf1, *sems):
    wid = lax.axis_index("s") * NUM_CORES + lax.axis_index("c")
    base = wid * ROWS_PER_WORKER
    bufs = (buf0, buf1)
    gsems = sems[:NBUF]
    ssems = sems[NBUF:]

    def issue_gather(i):
        return pltpu.async_copy(
            pe_hbm.at[pl.ds(base + OFFS[i], CHUNKS[i])],
            bufs[i % NBUF].at[pl.ds(0, CHUNKS[i])],
            gsems[i % NBUF])

    def issue_scatter(i):
        return pltpu.async_copy(
            bufs[i % NBUF].at[pl.ds(0, CHUNKS[i])],
            out_hbm.at[pl.ds(base + OFFS[i], CHUNKS[i])],
            ssems[i % NBUF])

    gath = [None] * NCHUNKS
    scat = [None] * NCHUNKS
    gath[0] = issue_gather(0)
    for i in range(NCHUNKS):
        if i + 1 < NCHUNKS:
            if i + 1 >= NBUF:
                scat[i + 1 - NBUF].wait()
            gath[i + 1] = issue_gather(i + 1)
        gath[i].wait()
        scat[i] = issue_scatter(i)
    for i in range(max(0, NCHUNKS - NBUF), NCHUNKS):
        scat[i].wait()


def kernel(x, pe):
    del x
    return _pe_lookup(pe)


# speedup vs baseline: 1.1385x; 1.0096x over previous
import functools

import jax
import jax.numpy as jnp
from jax import lax
from jax.experimental import pallas as pl
from jax.experimental.pallas import tpu as pltpu
import jax.experimental.pallas.tpu_sc as plsc

ROWS = 8192
DIM = 1024
NUM_CORES = 2
NUM_SUBCORES = 16
NUM_WORKERS = NUM_CORES * NUM_SUBCORES
ROWS_PER_WORKER = ROWS // NUM_WORKERS
CHUNKS = [16, 64, 56, 64, 40, 16]
assert sum(CHUNKS) == ROWS_PER_WORKER
OFFS = [sum(CHUNKS[:i]) for i in range(len(CHUNKS))]
NCHUNKS = len(CHUNKS)
NBUF = 2
BUFROWS = (56, 64)


@functools.partial(
    pl.kernel,
    out_type=jax.ShapeDtypeStruct((ROWS, DIM), jnp.float32),
    mesh=plsc.VectorSubcoreMesh(core_axis_name="c", subcore_axis_name="s"),
    scratch_types=(
        [pltpu.VMEM((BUFROWS[0], DIM), jnp.float32),
         pltpu.VMEM((BUFROWS[1], DIM), jnp.float32)]
        + [pltpu.SemaphoreType.DMA] * (2 * NBUF)
    ),
)
def _pe_lookup(pe_hbm, out_hbm, buf0, buf1, *sems):
    wid = lax.axis_index("s") * NUM_CORES + lax.axis_index("c")
    base = wid * ROWS_PER_WORKER
    bufs = (buf0, buf1)
    gsems = sems[:NBUF]
    ssems = sems[NBUF:]

    def issue_gather(i):
        return pltpu.async_copy(
            pe_hbm.at[pl.ds(base + OFFS[i], CHUNKS[i])],
            bufs[i % NBUF].at[pl.ds(0, CHUNKS[i])],
            gsems[i % NBUF])

    def issue_scatter(i):
        return pltpu.async_copy(
            bufs[i % NBUF].at[pl.ds(0, CHUNKS[i])],
            out_hbm.at[pl.ds(base + OFFS[i], CHUNKS[i])],
            ssems[i % NBUF])

    gath = [None] * NCHUNKS
    scat = [None] * NCHUNKS
    gath[0] = issue_gather(0)
    for i in range(NCHUNKS):
        if i + 1 < NCHUNKS:
            if i + 1 >= NBUF:
                scat[i + 1 - NBUF].wait()
            gath[i + 1] = issue_gather(i + 1)
        gath[i].wait()
        scat[i] = issue_scatter(i)
    for i in range(max(0, NCHUNKS - NBUF), NCHUNKS):
        scat[i].wait()


def kernel(x, pe):
    del x
    return _pe_lookup(pe)
